# Initial kernel scaffold; baseline (speedup 1.0000x reference)
import jax
import jax.numpy as jnp
from jax.experimental import pallas as pl


def kernel(token_ids, embedding_matrix):
    raise NotImplementedError("write your pallas kernel here")


# speedup vs baseline: 1.1032x; 1.1032x over previous
import functools

import jax
import jax.numpy as jnp
from jax import lax
from jax.experimental import pallas as pl
from jax.experimental.pallas import tpu as pltpu
from jax.experimental.pallas import tpu_sc as plsc

_CHUNK = 1600


@functools.lru_cache(maxsize=None)
def _build_gather(B, V, D):
    info = plsc.get_sparse_core_info()
    NC, NS = info.num_cores, info.num_subcores
    NW = NC * NS
    assert B % NW == 0
    b_per_w = B // NW
    ch = min(_CHUNK, b_per_w)
    assert b_per_w % ch == 0 and ch % 8 == 0
    n_chunks = b_per_w // ch

    mesh = plsc.VectorSubcoreMesh(core_axis_name="c", subcore_axis_name="s")

    @functools.partial(
        pl.kernel,
        mesh=mesh,
        out_type=jax.ShapeDtypeStruct((B, D), jnp.float32),
        scratch_types=[
            pltpu.VMEM((ch,), jnp.int32),
            pltpu.VMEM((ch, D), jnp.float32),
            pltpu.SemaphoreType.DMA,
        ],
        compiler_params=pltpu.CompilerParams(use_tc_tiling_on_sc=False),
    )
    def gather_kernel(idx_hbm, table_hbm, out_hbm, idx_v, rows_v, sem):
        wid = lax.axis_index("s") * NC + lax.axis_index("c")
        base_w = wid * b_per_w
        for c in range(n_chunks):
            base = base_w + c * ch
            pltpu.sync_copy(idx_hbm.at[pl.ds(base, ch)], idx_v)
            pltpu.async_copy(table_hbm.at[idx_v], rows_v, sem).wait()
            pltpu.sync_copy(rows_v, out_hbm.at[pl.ds(base, ch)])

    return gather_kernel


def kernel(token_ids, embedding_matrix):
    S0, S1 = token_ids.shape
    V, D = embedding_matrix.shape
    B = S0 * S1
    idx = token_ids.reshape(B).astype(jnp.int32)
    out = _build_gather(B, V, D)(idx, embedding_matrix)
    return out.reshape(S0, S1, D)


# speedup vs baseline: 1.1101x; 1.0063x over previous
import functools

import jax
import jax.numpy as jnp
from jax import lax
from jax.experimental import pallas as pl
from jax.experimental.pallas import tpu as pltpu
from jax.experimental.pallas import tpu_sc as plsc

_CHUNK = 1280


@functools.lru_cache(maxsize=None)
def _build_gather(B, V, D):
    info = plsc.get_sparse_core_info()
    NC, NS = info.num_cores, info.num_subcores
    NW = NC * NS
    assert B % NW == 0
    b_per_w = B // NW
    ch = min(_CHUNK, b_per_w)
    assert b_per_w % ch == 0 and ch % 8 == 0
    n_chunks = b_per_w // ch

    mesh = plsc.VectorSubcoreMesh(core_axis_name="c", subcore_axis_name="s")

    @functools.partial(
        pl.kernel,
        mesh=mesh,
        out_type=jax.ShapeDtypeStruct((B, D), jnp.float32),
        scratch_types=[
            pltpu.VMEM((b_per_w,), jnp.int32),
            pltpu.VMEM((2, ch, D), jnp.float32),
            pltpu.SemaphoreType.DMA((2,)),
            pltpu.SemaphoreType.DMA((2,)),
        ],
        compiler_params=pltpu.CompilerParams(use_tc_tiling_on_sc=False),
    )
    def gather_kernel(idx_hbm, table_hbm, out_hbm, idx_v, rows_v, gsem, wsem):
        wid = lax.axis_index("s") * NC + lax.axis_index("c")
        base_w = wid * b_per_w
        pltpu.sync_copy(idx_hbm.at[pl.ds(base_w, b_per_w)], idx_v)

        def start_gather(c):
            b = c % 2
            return pltpu.async_copy(
                table_hbm.at[idx_v.at[pl.ds(c * ch, ch)]],
                rows_v.at[b],
                gsem.at[b],
            )

        gathers = [None] * n_chunks
        writes = [None] * n_chunks
        gathers[0] = start_gather(0)
        for c in range(n_chunks):
            b = c % 2
            gathers[c].wait()
            writes[c] = pltpu.async_copy(
                rows_v.at[b],
                out_hbm.at[pl.ds(base_w + c * ch, ch)],
                wsem.at[b],
            )
            if c + 1 < n_chunks:
                if c >= 1:
                    writes[c - 1].wait()
                gathers[c + 1] = start_gather(c + 1)
        writes[n_chunks - 2].wait()
        writes[n_chunks - 1].wait()

    return gather_kernel


def kernel(token_ids, embedding_matrix):
    S0, S1 = token_ids.shape
    V, D = embedding_matrix.shape
    B = S0 * S1
    idx = token_ids.reshape(B).astype(jnp.int32)
    out = _build_gather(B, V, D)(idx, embedding_matrix)
    return out.reshape(S0, S1, D)


# speedup vs baseline: 1.6134x; 1.4533x over previous
import functools

import jax
import jax.numpy as jnp
from jax import lax
from jax.experimental import pallas as pl
from jax.experimental.pallas import tpu as pltpu
from jax.experimental.pallas import tpu_sc as plsc


@functools.lru_cache(maxsize=None)
def _build(B, S, V, D):
    info = plsc.get_sparse_core_info()
    NC, NS, L = info.num_cores, info.num_subcores, info.num_lanes
    NW = NC * NS
    BLK = 128
    n_blocks = B // BLK
    blocks_per_w = n_blocks // NW
    assert n_blocks % NW == 0 and D == 32 and L == 16
    n_units = blocks_per_w * S
    groups = BLK // L

    mesh = plsc.VectorSubcoreMesh(core_axis_name="c", subcore_axis_name="s")

    @functools.partial(
        pl.kernel,
        mesh=mesh,
        out_type=jax.ShapeDtypeStruct((S, D, B), jnp.float32),
        scratch_types=[
            pltpu.VMEM((BLK * S,), jnp.int32),
            pltpu.VMEM((2, BLK), jnp.int32),
            pltpu.VMEM((2, BLK), jnp.int32),
            pltpu.VMEM((2, BLK, 128), jnp.float32),
            pltpu.VMEM((2, D, BLK), jnp.float32),
            pltpu.SemaphoreType.DMA((2,)),
            pltpu.SemaphoreType.DMA((2,)),
        ],
        compiler_params=pltpu.CompilerParams(
            use_tc_tiling_on_sc=True, needs_layout_passes=False),
    )
    def gather_kernel(idx_hbm, tab_hbm, out_hbm, idxblk, iv2, qv, v, w, gsem, wsem):
        wid = lax.axis_index("s") * NC + lax.axis_index("c")
        blk0 = wid * blocks_per_w
        iotas = [(lax.iota(jnp.int32, L) + kb * L) * S for kb in range(groups)]
        lane_ids = [lax.iota(jnp.int32, L) + kb * L for kb in range(groups)]

        def load_idxblk(blk):
            pltpu.sync_copy(idx_hbm.at[pl.ds((blk0 + blk) * BLK * S, BLK * S)], idxblk)

        def prep(s, slot):
            for kb in range(groups):
                orig = plsc.load_gather(idxblk, [iotas[kb] + s])
                iv2[slot, pl.ds(kb * L, L)] = lax.shift_right_logical(orig, 2)
                qv[slot, pl.ds(kb * L, L)] = lax.shift_left(
                    lax.bitwise_and(orig, 3), 5)

        def start_gather(slot):
            return pltpu.async_copy(tab_hbm.at[iv2.at[slot]], v.at[slot], gsem.at[slot])

        def wait_gather(slot):
            pltpu.make_async_copy(tab_hbm.at[iv2.at[slot]], v.at[slot], gsem.at[slot]).wait()

        def transpose(slot):
            for kb in range(groups):
                qcol = qv[slot, pl.ds(kb * L, L)]
                for d in range(D):
                    val = plsc.load_gather(
                        v, [jnp.full((L,), slot, jnp.int32), lane_ids[kb], qcol + d])
                    w[slot, d, pl.ds(kb * L, L)] = val

        def start_write(slot, s, blk):
            return pltpu.async_copy(
                w.at[slot],
                out_hbm.at[s, :, pl.ds((blk0 + blk) * BLK, BLK)],
                wsem.at[slot],
            )

        def wait_write(slot, s, blk):
            pltpu.make_async_copy(
                w.at[slot],
                out_hbm.at[s, :, pl.ds((blk0 + blk) * BLK, BLK)],
                wsem.at[slot],
            ).wait()

        load_idxblk(0)
        prep(0, 0)
        start_gather(0)

        def body(u2, carry):
            for j in (0, 1):
                u = 2 * u2 + j
                s = lax.rem(u, S)
                blk = lax.div(u, S)
                u1 = u + 1
                s1 = lax.rem(u1, S)
                blk1 = lax.div(u1, S)

                wait_gather(j)

                @pl.when(jnp.logical_and(u1 < n_units, s1 == 0))
                def _():
                    load_idxblk(blk1)

                prep(s1, 1 - j)

                @pl.when(u1 < n_units)
                def _():
                    start_gather(1 - j)

                @pl.when(u2 >= 1)
                def _():
                    wait_write(j, s, blk)

                transpose(j)
                start_write(j, s, blk)
            return carry

        lax.fori_loop(0, n_units // 2, body, 0)
        wait_write(0, 0, 0)
        wait_write(1, 0, 0)

    return gather_kernel


def kernel(token_ids, embedding_matrix):
    S0, S1 = token_ids.shape
    V, D = embedding_matrix.shape
    B = S0 * S1
    idx = token_ids.reshape(B)
    if idx.dtype != jnp.int32:
        idx = idx.astype(jnp.int32)
    t2 = embedding_matrix.reshape(V // 4, D * 4)
    out = _build(S0, S1, V, D)(idx, t2)
    return jnp.transpose(out, (2, 0, 1))
